# baseline (speedup 1.0000x reference)
import functools

import jax
import jax.numpy as jnp
from jax import lax
from jax.experimental import pallas as pl
from jax.experimental.pallas import tpu as pltpu
from jax.experimental.pallas import tpu_sc as plsc

NC, NS, L = 2, 16, 16
NW = NC * NS
ROWS, COLS = 16384, 360
RPW = ROWS // NW
CH = 64
NCHUNK = RPW // CH
GROUPS = CH // L
NPAIR = 11
TOFF = COLS - L
SSTR = L + 1

_mesh = plsc.VectorSubcoreMesh(core_axis_name="c", subcore_axis_name="s")


@functools.partial(
    pl.kernel,
    mesh=_mesh,
    compiler_params=pltpu.CompilerParams(needs_layout_passes=False,
                                         use_tc_tiling_on_sc=True),
    out_type=jax.ShapeDtypeStruct((ROWS,), jnp.float32),
    scratch_types=[
        pltpu.VMEM((CH, COLS), jnp.float32),
        pltpu.VMEM((CH, COLS), jnp.float32),
        pltpu.VMEM((L * SSTR,), jnp.float32),
        pltpu.VMEM((L * SSTR,), jnp.int32),
        pltpu.VMEM((COLS,), jnp.int32),
        pltpu.VMEM((RPW,), jnp.float32),
        pltpu.SemaphoreType.DMA,
        pltpu.SemaphoreType.DMA,
    ],
)
def _argmax_deg_kernel(in_hbm, degs_hbm, out_hbm, buf0, buf1, vals_v, cols_v,
                       degs_v, out_v, sem0, sem1):
    wid = lax.axis_index("s") * NC + lax.axis_index("c")
    base_row = wid * RPW

    pltpu.sync_copy(degs_hbm, degs_v)

    iota = lax.iota(jnp.int32, L)
    i17 = iota * SSTR
    neg_inf = jnp.full((L,), -jnp.inf, jnp.float32)
    zero = jnp.zeros((L,), jnp.int32)
    sixteen = jnp.full((L,), L, jnp.int32)

    bufs = [buf0, buf1]
    sems = [sem0, sem1]
    copies = [None, None]

    def start(ci, b):
        src = in_hbm.at[pl.ds(base_row + ci * CH, CH)]
        copies[b] = pltpu.async_copy(src, bufs[b], sems[b])

    start(0, 0)
    for ci in range(NCHUNK):
        b = ci & 1
        if ci + 1 < NCHUNK:
            start(ci + 1, 1 - b)
        copies[b].wait()
        buf = bufs[b]

        def group_body(g, _):
            def rowpair_body(r2, _):
                r0 = 2 * r2
                rows = [g * L + r0, g * L + r0 + 1]
                best = [neg_inf] * 2
                bbase = [zero] * 2
                for q in range(NPAIR):
                    off = q * 2 * L
                    offv = jnp.full((L,), off, jnp.int32)
                    m = [jnp.maximum(buf[rows[k], pl.ds(off, L)],
                                     buf[rows[k], pl.ds(off + L, L)])
                         for k in range(2)]
                    p = [m[k] > best[k] for k in range(2)]
                    best = [jnp.maximum(m[k], best[k]) for k in range(2)]
                    bbase = [jnp.where(p[k], offv, bbase[k])
                             for k in range(2)]
                toffv = jnp.full((L,), TOFF, jnp.int32)
                v = [buf[rows[k], pl.ds(TOFF, L)] for k in range(2)]
                p = [v[k] > best[k] for k in range(2)]
                best = [jnp.maximum(v[k], best[k]) for k in range(2)]
                bbase = [jnp.where(p[k], toffv, bbase[k]) for k in range(2)]
                for k in range(2):
                    idx = bbase[k] + iota
                    rowv = jnp.full((L,), rows[k], jnp.int32)
                    ve = plsc.load_gather(buf, [rowv, idx])
                    col = idx + jnp.where(ve == best[k], zero, sixteen)
                    vals_v[pl.ds((r0 + k) * SSTR, L)] = best[k]
                    cols_v[pl.ds((r0 + k) * SSTR, L)] = col
                return 0

            lax.fori_loop(0, L // 2, rowpair_body, 0)

            best = neg_inf
            bcol = zero
            for j in range(L):
                v = plsc.load_gather(vals_v, [i17 + j if j else i17])
                cj = plsc.load_gather(cols_v, [i17 + j if j else i17])
                pg = v > best
                pe = (v == best) & (cj < bcol)
                p = pg | pe
                best = jnp.where(p, v, best)
                bcol = jnp.where(p, cj, bcol)
            d = plsc.load_gather(degs_v, [bcol])
            out_v[pl.ds(ci * CH + g * L, L)] = d.astype(jnp.float32)
            return 0

        lax.fori_loop(0, GROUPS, group_body, 0)

    pltpu.sync_copy(out_v, out_hbm.at[pl.ds(base_row, RPW)])


@jax.jit
def kernel(inputs, degs):
    return _argmax_deg_kernel(inputs, degs)


# speedup vs baseline: 1.6345x; 1.6345x over previous
import functools

import jax
import jax.numpy as jnp
from jax import lax
from jax.experimental import pallas as pl
from jax.experimental.pallas import tpu as pltpu
from jax.experimental.pallas import tpu_sc as plsc

NC, NS, L = 2, 16, 16
NW = NC * NS
ROWS, COLS = 16384, 360
RPW = ROWS // NW
CC = 128
NCH = RPW // CC
GROUPS = CC // L
QUADS = COLS // 4

_mesh = plsc.VectorSubcoreMesh(core_axis_name="c", subcore_axis_name="s")


@functools.partial(
    pl.kernel,
    mesh=_mesh,
    compiler_params=pltpu.CompilerParams(needs_layout_passes=False,
                                         use_tc_tiling_on_sc=True),
    out_type=jax.ShapeDtypeStruct((ROWS,), jnp.float32),
    scratch_types=[
        pltpu.VMEM((COLS, CC), jnp.float32),
        pltpu.VMEM((COLS, CC), jnp.float32),
        pltpu.VMEM((COLS,), jnp.int32),
        pltpu.VMEM((RPW,), jnp.float32),
        pltpu.SemaphoreType.DMA,
        pltpu.SemaphoreType.DMA,
    ],
)
def _argmax_deg_kernel(xt_hbm, degs_hbm, out_hbm, buf0, buf1, degs_v, out_v,
                       sem0, sem1):
    wid = lax.axis_index("s") * NC + lax.axis_index("c")
    col_base = wid * RPW

    pltpu.sync_copy(degs_hbm, degs_v)

    iota = lax.iota(jnp.int32, L)
    neg_inf = jnp.full((L,), -jnp.inf, jnp.float32)
    zero = jnp.zeros((L,), jnp.int32)
    one = jnp.full((L,), 1, jnp.int32)
    two = jnp.full((L,), 2, jnp.int32)
    three = jnp.full((L,), 3, jnp.int32)

    bufs = [buf0, buf1]
    sems = [sem0, sem1]
    copies = [None, None]

    def start(ci, b):
        src = xt_hbm.at[:, pl.ds(col_base + ci * CC, CC)]
        copies[b] = pltpu.async_copy(src, bufs[b], sems[b])

    start(0, 0)
    for ci in range(NCH):
        b = ci & 1
        if ci + 1 < NCH:
            start(ci + 1, 1 - b)
        copies[b].wait()
        buf = bufs[b]

        for gp in range(GROUPS // 2):
            c0 = [(2 * gp) * L, (2 * gp + 1) * L]

            def quad_body(q, carry):
                b0, q0, b1, q1 = carry
                r = 4 * q
                qv = jnp.full((L,), q, jnp.int32)
                m = [jnp.maximum(
                        jnp.maximum(buf[r, pl.ds(c0[k], L)],
                                    buf[r + 1, pl.ds(c0[k], L)]),
                        jnp.maximum(buf[r + 2, pl.ds(c0[k], L)],
                                    buf[r + 3, pl.ds(c0[k], L)]))
                     for k in range(2)]
                p0 = m[0] > b0
                p1 = m[1] > b1
                b0 = jnp.maximum(m[0], b0)
                b1 = jnp.maximum(m[1], b1)
                q0 = jnp.where(p0, qv, q0)
                q1 = jnp.where(p1, qv, q1)
                return b0, q0, b1, q1

            b0, q0, b1, q1 = lax.fori_loop(
                0, QUADS, quad_body, (neg_inf, zero, neg_inf, zero))

            for k, (best, bq) in enumerate(((b0, q0), (b1, q1))):
                r0 = bq * 4
                lanes = c0[k] + iota
                v0 = plsc.load_gather(buf, [r0, lanes])
                v1 = plsc.load_gather(buf, [r0 + one, lanes])
                v2 = plsc.load_gather(buf, [r0 + two, lanes])
                row = r0 + jnp.where(
                    v0 == best, zero,
                    jnp.where(v1 == best, one,
                              jnp.where(v2 == best, two, three)))
                d = plsc.load_gather(degs_v, [row])
                out_v[pl.ds(ci * CC + c0[k], L)] = d.astype(jnp.float32)

    pltpu.sync_copy(out_v, out_hbm.at[pl.ds(col_base, RPW)])


@jax.jit
def kernel(inputs, degs):
    return _argmax_deg_kernel(inputs.T, degs)
